# speedup vs baseline: 1.2478x; 1.2478x over previous
import functools

import jax
import jax.numpy as jnp
from jax.experimental import pallas as pl
from jax.experimental.pallas import tpu as pltpu

N_EXP = 8
K = 2
HID = 1024
INTER = 2048
TOK = 2048
PAIRS = TOK * K

BLK = 128
PAD_ROWS = PAIRS + N_EXP * BLK
NB = PAD_ROWS // BLK


def _ffn_body(be_ref, x_ref, w0_ref, w1_ref, wo_ref, ws_ref, y_ref):
    x = x_ref[...]
    h0 = jnp.dot(x, w0_ref[0], preferred_element_type=jnp.float32)
    h1 = jnp.dot(x, w1_ref[0], preferred_element_type=jnp.float32)
    act = (h0 * jax.nn.sigmoid(h0)) * h1
    y = jnp.dot(act, wo_ref[0], preferred_element_type=jnp.float32)
    y_ref[...] = y * ws_ref[...]


def _grouped_ffn(x_sorted, wi_0, wi_1, wo, w_sorted, block_expert):
    grid_spec = pltpu.PrefetchScalarGridSpec(
        num_scalar_prefetch=1,
        grid=(NB,),
        in_specs=[
            pl.BlockSpec((BLK, HID), lambda b, be: (b, 0)),
            pl.BlockSpec((1, HID, INTER), lambda b, be: (be[b], 0, 0)),
            pl.BlockSpec((1, HID, INTER), lambda b, be: (be[b], 0, 0)),
            pl.BlockSpec((1, INTER, HID), lambda b, be: (be[b], 0, 0)),
            pl.BlockSpec((BLK, 1), lambda b, be: (b, 0)),
        ],
        out_specs=pl.BlockSpec((BLK, HID), lambda b, be: (b, 0)),
    )
    return pl.pallas_call(
        _ffn_body,
        grid_spec=grid_spec,
        out_shape=jax.ShapeDtypeStruct((PAD_ROWS, HID), jnp.float32),
    )(block_expert, x_sorted, wi_0, wi_1, wo, w_sorted.reshape(PAD_ROWS, 1))


def kernel(inputs, router_logits, wi_0, wi_1, wo):
    top_logits, top_idx = jax.lax.top_k(router_logits, K)
    w = jax.nn.softmax(top_logits.astype(jnp.float32), axis=-1)
    e_flat = top_idx.reshape(-1).astype(jnp.int32)
    w_flat = w.reshape(-1)

    sort_idx = jnp.argsort(e_flat, stable=True)
    e_sorted = e_flat[sort_idx]
    cnt = jnp.sum(jax.nn.one_hot(e_flat, N_EXP, dtype=jnp.int32), axis=0)
    padded = ((cnt + BLK - 1) // BLK) * BLK
    poff = jnp.concatenate([jnp.zeros(1, jnp.int32), jnp.cumsum(padded)])
    off = jnp.concatenate([jnp.zeros(1, jnp.int32), jnp.cumsum(cnt)])
    ranks = jnp.arange(PAIRS, dtype=jnp.int32)
    pos = poff[e_sorted] + (ranks - off[e_sorted])

    row_src = jnp.zeros(PAD_ROWS, jnp.int32).at[pos].set(sort_idx // K)
    w_sorted = jnp.zeros(PAD_ROWS, jnp.float32).at[pos].set(w_flat[sort_idx])
    pos_pair = jnp.zeros(PAIRS, jnp.int32).at[sort_idx].set(pos)
    pos0 = pos_pair[0::K]
    pos1 = pos_pair[1::K]

    block_rows = jnp.arange(NB, dtype=jnp.int32) * BLK
    block_expert = jnp.clip(
        jnp.searchsorted(poff, block_rows, side="right").astype(jnp.int32) - 1,
        0, N_EXP - 1)

    x_sorted = inputs[row_src]
    y = _grouped_ffn(x_sorted, wi_0, wi_1, wo, w_sorted, block_expert)
    return y[pos0] + y[pos1]
